# Initial kernel scaffold; baseline (speedup 1.0000x reference)
import jax
import jax.numpy as jnp
from jax.experimental import pallas as pl


def kernel(x, pe):
    raise NotImplementedError("write your pallas kernel here")


# speedup vs baseline: 9.5200x; 9.5200x over previous
import functools

import jax
import jax.numpy as jnp
from jax import lax
from jax.experimental import pallas as pl
from jax.experimental.pallas import tpu as pltpu
from jax.experimental.pallas import tpu_sc as plsc

EMBED = 128
SEQ = 512
PE_LEN = 1024
CHUNK = 16
WIN = 528


@functools.partial(jax.jit, static_argnums=(1,))
def _run(pe, batch):
    info = plsc.get_sparse_core_info()
    nw = info.num_cores * info.num_subcores
    assert SEQ % CHUNK == 0 and SEQ // CHUNK == nw
    mesh = plsc.VectorSubcoreMesh(core_axis_name="c", subcore_axis_name="s")

    @functools.partial(
        pl.kernel,
        mesh=mesh,
        out_type=jax.ShapeDtypeStruct((batch * SEQ * SEQ, EMBED), jnp.float32),
        scratch_types=[
            pltpu.VMEM((WIN, EMBED), jnp.float32),
            pltpu.SemaphoreType.DMA,
        ],
    )
    def body(pe_hbm, out_hbm, buf, sem):
        w = lax.axis_index("s") * info.num_cores + lax.axis_index("c")
        i0 = w * CHUNK
        lo = 496 - i0
        pltpu.sync_copy(pe_hbm.at[pl.ds(lo, WIN)], buf)
        handles = []
        for k in range(CHUNK):
            src = buf.at[pl.ds(CHUNK - k, SEQ)]
            for b in range(batch):
                row = (b * SEQ + i0 + k) * SEQ
                handles.append(pltpu.async_copy(src, out_hbm.at[pl.ds(row, SEQ)], sem))
        for h in handles:
            h.wait()

    return body(pe).reshape(batch, SEQ, SEQ, EMBED)


def kernel(x, pe):
    return _run(pe, x.shape[0])


# speedup vs baseline: 9.7800x; 1.0273x over previous
import functools

import jax
import jax.numpy as jnp
from jax import lax
from jax.experimental import pallas as pl
from jax.experimental.pallas import tpu as pltpu
from jax.experimental.pallas import tpu_sc as plsc

EMBED = 128
SEQ = 512
PE_LEN = 1024
CHUNK = 16
WIN = 528
TILE_K = 10


@functools.partial(jax.jit, static_argnums=(1,))
def _run(pe, batch):
    info = plsc.get_sparse_core_info()
    nw = info.num_cores * info.num_subcores
    assert SEQ % CHUNK == 0 and SEQ // CHUNK == nw
    mesh = plsc.VectorSubcoreMesh(core_axis_name="c", subcore_axis_name="s")

    @functools.partial(
        pl.kernel,
        mesh=mesh,
        out_type=jax.ShapeDtypeStruct((batch * SEQ * SEQ, EMBED), jnp.float32),
        scratch_types=[
            pltpu.VMEM((WIN, EMBED), jnp.float32),
            pltpu.VMEM_SHARED((PE_LEN, EMBED), jnp.float32),
            pltpu.SemaphoreType.DMA,
            pltpu.SemaphoreType.DMA,
        ],
    )
    def body(pe_hbm, out_hbm, buf, pe_sh, sem, sem2):
        s = lax.axis_index("s")
        w = s * info.num_cores + lax.axis_index("c")
        i0 = w * CHUNK
        lo = 496 - i0
        win = pltpu.async_copy(pe_hbm.at[pl.ds(lo, WIN)], buf, sem2)
        @pl.when(s == 0)
        def _():
            pltpu.sync_copy(pe_hbm, pe_sh)
        plsc.subcore_barrier()
        win.wait()
        handles = []
        for k in range(CHUNK):
            for b in range(batch):
                row = (b * SEQ + i0 + k) * SEQ
                dst = out_hbm.at[pl.ds(row, SEQ)]
                if k < TILE_K:
                    src = buf.at[pl.ds(CHUNK - k, SEQ)]
                else:
                    src = pe_sh.at[pl.ds(SEQ - (i0 + k), SEQ)]
                handles.append(pltpu.async_copy(src, dst, sem))
        for h in handles:
            h.wait()

    return body(pe).reshape(batch, SEQ, SEQ, EMBED)


def kernel(x, pe):
    return _run(pe, x.shape[0])


# speedup vs baseline: 9.7995x; 1.0020x over previous
import functools

import jax
import jax.numpy as jnp
from jax import lax
from jax.experimental import pallas as pl
from jax.experimental.pallas import tpu as pltpu
from jax.experimental.pallas import tpu_sc as plsc

EMBED = 128
SEQ = 512
PE_LEN = 1024
CHUNK = 16
WIN = 528
TILE_K = 10


@functools.partial(jax.jit, static_argnums=(1,))
def _run(pe, batch):
    info = plsc.get_sparse_core_info()
    nw = info.num_cores * info.num_subcores
    assert SEQ % CHUNK == 0 and SEQ // CHUNK == nw
    mesh = plsc.VectorSubcoreMesh(core_axis_name="c", subcore_axis_name="s")

    @functools.partial(
        pl.kernel,
        mesh=mesh,
        out_type=jax.ShapeDtypeStruct((batch * SEQ * SEQ, EMBED), jnp.float32),
        scratch_types=[
            pltpu.VMEM((WIN, EMBED), jnp.float32),
            pltpu.VMEM_SHARED((PE_LEN, EMBED), jnp.float32),
            pltpu.SemaphoreType.DMA,
            pltpu.SemaphoreType.DMA,
        ],
    )
    def body(pe_hbm, out_hbm, buf, pe_sh, sem, sem2):
        s = lax.axis_index("s")
        w = s * info.num_cores + lax.axis_index("c")
        i0 = w * CHUNK
        lo = 496 - i0
        win = pltpu.async_copy(pe_hbm.at[pl.ds(lo, WIN)], buf, sem2)
        @pl.when(s == 0)
        def _():
            pltpu.sync_copy(pe_hbm, pe_sh)
        plsc.subcore_barrier()
        win.wait()
        handles = []
        for k in range(CHUNK):
            for b in range(batch):
                row = (b * SEQ + i0 + k) * SEQ
                dst = out_hbm.at[pl.ds(row, SEQ)]
                if k % 8 < TILE_K // 2:
                    src = buf.at[pl.ds(CHUNK - k, SEQ)]
                else:
                    src = pe_sh.at[pl.ds(SEQ - (i0 + k), SEQ)]
                handles.append(pltpu.async_copy(src, dst, sem))
        for h in handles:
            h.wait()

    return body(pe).reshape(batch, SEQ, SEQ, EMBED)


def kernel(x, pe):
    return _run(pe, x.shape[0])
